# baseline (speedup 1.0000x reference)
import functools

import jax
import jax.numpy as jnp
from jax import lax
from jax.experimental import pallas as pl
from jax.experimental.pallas import tpu as pltpu
from jax.experimental.pallas import tpu_sc as plsc

NC = 2
NS = 16
NW = NC * NS
CH = 128
GRP = 3
BODY = 6
LRELU = 0.1
BN_EPS = 1e-5
TRASH = 16


def _chunks_per_worker(E):
    n = -(-E // (NW * CH))
    return -(-n // BODY) * BODY


def _make_sc_spmm(N, D, E, with_deg):
    npc = _chunks_per_worker(E)
    gpw = npc // GRP
    NP = NS * 640

    mesh = plsc.VectorSubcoreMesh(
        core_axis_name="c", subcore_axis_name="s",
        num_cores=NC, num_subcores=NS)

    out_type = [jax.ShapeDtypeStruct((NC, N, D), jnp.float32)]
    if with_deg:
        out_type.append(jax.ShapeDtypeStruct((NC, NP), jnp.float32))

    scratch = [
        pltpu.VMEM_SHARED((N + TRASH, D), jnp.float32),
        pltpu.VMEM_SHARED((NP,), jnp.float32),
        pltpu.VMEM((GRP * 18, 128), jnp.float32),
        pltpu.VMEM((GRP * 18, 128), jnp.float32),
        pltpu.VMEM((2, 128), jnp.int32),
        pltpu.VMEM((2, 128), jnp.int32),
        pltpu.VMEM((CH, D), jnp.float32),
        pltpu.VMEM((CH, D), jnp.float32),
        pltpu.VMEM((1, GRP * 128), jnp.int32),
        pltpu.VMEM((1, GRP * 128), jnp.int32),
        pltpu.VMEM((GRP * 128,), jnp.float32),
        pltpu.VMEM((640,), jnp.float32),
        pltpu.SemaphoreType.DMA,
        pltpu.SemaphoreType.DMA,
        pltpu.SemaphoreType.DMA,
        pltpu.SemaphoreType.DMA,
        pltpu.SemaphoreType.DMA,
        pltpu.SemaphoreType.DMA,
        pltpu.SemaphoreType.DMA,
        pltpu.SemaphoreType.DMA,
    ]

    @functools.partial(pl.kernel, out_type=tuple(out_type), mesh=mesh,
                       scratch_types=scratch)
    def spmm(*refs):
        if with_deg:
            (h_hbm, meta_hbm, z2_hbm, out_hbm, deg_hbm, acc, deg_sh,
             meta0, meta1, idx0, idx1, rows0, rows1, dct0, dct1,
             ones_v, zer_v,
             sem_m0, sem_m1, sem_g0, sem_g1, sem_s0, sem_s1,
             sem_d0, sem_d1) = refs
        else:
            (h_hbm, meta_hbm, z2_hbm, out_hbm, acc, deg_sh,
             meta0, meta1, idx0, idx1, rows0, rows1, dct0, dct1,
             ones_v, zer_v,
             sem_m0, sem_m1, sem_g0, sem_g1, sem_s0, sem_s1,
             sem_d0, sem_d1) = refs
            deg_hbm = None

        c = lax.axis_index("c")
        s = lax.axis_index("s")
        wid = s * NC + c
        meta = (meta0, meta1)
        idx = (idx0, idx1)
        rows = (rows0, rows1)
        dct = (dct0, dct1)
        sem_m = (sem_m0, sem_m1)
        sem_g = (sem_g0, sem_g1)
        sem_s = (sem_s0, sem_s1)
        sem_d = (sem_d0, sem_d1)

        @pl.when(s < NS - 1)
        def _():
            pltpu.sync_copy(z2_hbm.at[pl.ds(s * 640, 640)],
                            acc.at[pl.ds(s * 640, 640)])

        @pl.when(s == NS - 1)
        def _():
            pltpu.sync_copy(z2_hbm.at[pl.ds(9600, 400)],
                            acc.at[pl.ds(9600, 400)])

        if with_deg:
            for i in range(640 // 16):
                zer_v[pl.ds(i * 16, 16)] = jnp.zeros((16,), jnp.float32)
            pltpu.sync_copy(zer_v, deg_sh.at[pl.ds(s * 640, 640)])
            for i in range(GRP * 128 // 16):
                ones_v[pl.ds(i * 16, 16)] = jnp.ones((16,), jnp.float32)

        plsc.subcore_barrier()

        g0 = wid * gpw
        glast = g0 + gpw - 1

        def meta_start(slot, gi):
            gi = jnp.minimum(gi, glast)
            pltpu.async_copy(meta_hbm.at[gi], meta[slot], sem_m[slot])

        def meta_wait(slot):
            pltpu.make_async_copy(meta_hbm.at[g0], meta[slot],
                                  sem_m[slot]).wait()

        def cvt(o1):
            p, ms, mr, gp = o1 % 2, (o1 // GRP) % 2, 18 * (o1 % GRP), o1 % GRP
            for k in range(128 // 16):
                sl = pl.ds(k * 16, 16)
                idx[p][0, sl] = meta[ms][mr, sl].astype(jnp.int32)
                dv = meta[ms][mr + 1, sl].astype(jnp.int32)
                idx[p][1, sl] = dv
                if with_deg:
                    dct[ms][0, pl.ds(gp * 128 + k * 16, 16)] = dv

        def gather_start(p):
            pltpu.async_copy(h_hbm.at[idx[p].at[0]], rows[p], sem_g[p])

        def gather_wait(p):
            pltpu.make_async_copy(h_hbm.at[idx[p].at[0]], rows[p],
                                  sem_g[p]).wait()

        def scale(o):
            p, ms, mr = o % 2, (o // GRP) % 2, 18 * (o % GRP)
            rp, mp = rows[p], meta[ms]

            def grp_fn(g, carry2):
                for j in range(16):
                    e = g * 16 + j
                    bc = mp[mr + 2 + 2 * g + (j // 8),
                            pl.ds((j % 8) * 16, 16)]
                    for k in range(D // 16):
                        sl = pl.ds(k * 16, 16)
                        rp[e, sl] = rp[e, sl] * bc
                return carry2

            lax.fori_loop(0, CH // 16, grp_fn, 0)

        def scatter_start(p):
            pltpu.async_copy(rows[p], acc.at[idx[p].at[1]], sem_s[p],
                             add=True)

        def scatter_wait(p):
            pltpu.make_async_copy(rows[p], acc.at[idx[p].at[1]],
                                  sem_s[p]).wait()

        def deg_start(ms):
            pltpu.async_copy(ones_v, deg_sh.at[dct[ms].at[0]], sem_d[ms],
                             add=True)

        def deg_wait(ms):
            pltpu.make_async_copy(ones_v, deg_sh.at[dct[ms].at[0]],
                                  sem_d[ms]).wait()

        meta_start(0, g0)
        meta_start(1, g0 + 1)
        meta_wait(0)
        cvt(0)
        gather_start(0)

        def body(t, carry):
            gb = g0 + 2 * t
            for o in range(BODY):
                o1 = o + 1
                p, p1 = o % 2, (o + 1) % 2
                ms1 = ((o1 // GRP) % 2) if o1 < BODY else 0
                if o == 0:
                    @pl.when(t > 0)
                    def _():
                        scatter_wait(p1)
                else:
                    scatter_wait(p1)
                if o1 % GRP == 0:
                    meta_wait(ms1)
                if with_deg and o1 % GRP == 0:
                    if ms1 == 1:
                        @pl.when(t > 0)
                        def _():
                            deg_wait(1)
                    else:
                        deg_wait(0)
                cvt(o1 % BODY)
                if with_deg and o1 % GRP == GRP - 1:
                    deg_start(ms1)
                gather_start(p1)
                gather_wait(p)
                scale(o)
                scatter_start(p)
                if o == GRP - 1:
                    meta_start(0, gb + 2)
                if o == BODY - 1:
                    meta_start(1, gb + 3)
            return carry

        lax.fori_loop(0, npc // BODY, body, 0)
        scatter_wait(1)
        meta_wait(1)
        gather_wait(0)
        if with_deg:
            deg_wait(1)

        plsc.subcore_barrier()

        @pl.when(s < NS - 1)
        def _():
            pltpu.sync_copy(acc.at[pl.ds(s * 640, 640)],
                            out_hbm.at[c, pl.ds(s * 640, 640)])

        @pl.when(s == NS - 1)
        def _():
            pltpu.sync_copy(acc.at[pl.ds(9600, 400)],
                            out_hbm.at[c, pl.ds(9600, 400)])

        if with_deg:
            pltpu.sync_copy(deg_sh.at[pl.ds(s * 640, 640)],
                            deg_hbm.at[c, pl.ds(s * 640, 640)])

    return spmm


def _dense_layer(p, deg3, h, Wl, bl, Wr, gamma, beta, final):
    N, D = h.shape
    R = 2000
    grid = (N // R,)

    def body(p_ref, deg_ref, h_ref, wl_ref, bl_ref, wr_ref, g_ref, b_ref,
             o_ref):
        pb = p_ref[0] + p_ref[1]
        degb = deg_ref[0, :, 0] + deg_ref[1, :, 0]
        inv = 1.0 / jnp.maximum(degb, 1.0)
        agg = pb * inv[:, None]
        hh = (jnp.dot(agg, wl_ref[...], preferred_element_type=jnp.float32)
              + bl_ref[...]
              + jnp.dot(h_ref[...], wr_ref[...],
                        preferred_element_type=jnp.float32))
        if final:
            nrm = jnp.sqrt(jnp.sum(hh * hh, axis=1, keepdims=True))
            o_ref[...] = hh / jnp.maximum(nrm, 1e-12)
        else:
            scale = g_ref[...] * (1.0 / jnp.sqrt(1.0 + BN_EPS))
            hh = hh * scale + b_ref[...]
            o_ref[...] = jnp.where(hh >= 0, hh, LRELU * hh)

    return pl.pallas_call(
        body,
        grid=grid,
        in_specs=[
            pl.BlockSpec((NC, R, D), lambda i: (0, i, 0)),
            pl.BlockSpec((NC, R, 1), lambda i: (0, i, 0)),
            pl.BlockSpec((R, D), lambda i: (i, 0)),
            pl.BlockSpec((D, D), lambda i: (0, 0)),
            pl.BlockSpec((1, D), lambda i: (0, 0)),
            pl.BlockSpec((D, D), lambda i: (0, 0)),
            pl.BlockSpec((1, D), lambda i: (0, 0)),
            pl.BlockSpec((1, D), lambda i: (0, 0)),
        ],
        out_specs=pl.BlockSpec((R, D), lambda i: (i, 0)),
        out_shape=jax.ShapeDtypeStruct((N, D), jnp.float32),
    )(p, deg3, h, Wl, bl.reshape(1, D), Wr, gamma.reshape(1, D),
      beta.reshape(1, D))


def kernel(x, edge_index, edge_weight, W1l, b1l, W1r, W2l, b2l, W2r,
           W3l, b3l, W3r, g1, be1, g2, be2):
    N, D = x.shape
    E = edge_weight.shape[0]
    Ep = _chunks_per_worker(E) * NW * CH

    pad = Ep - E
    src = jnp.concatenate([edge_index[0], jnp.zeros((pad,), jnp.int32)])
    tr = N + (jnp.arange(pad, dtype=jnp.int32) % TRASH)
    dst = jnp.concatenate([edge_index[1], tr])
    w = jnp.concatenate([edge_weight, jnp.zeros((pad,), jnp.float32)])

    nck = Ep // CH
    w16 = jnp.repeat(w, 16).reshape(nck, 16, 128)
    meta = jnp.concatenate(
        [src.astype(jnp.float32).reshape(nck, 1, 128),
         dst.astype(jnp.float32).reshape(nck, 1, 128), w16],
        axis=1).reshape(nck // GRP, GRP * 18, 128)

    z2 = jnp.zeros((N, D), jnp.float32)

    spmm_deg = _make_sc_spmm(N, D, Ep, with_deg=True)
    spmm = _make_sc_spmm(N, D, Ep, with_deg=False)

    p1, deg = spmm_deg(x, meta, z2)
    deg3 = deg[:, :N].reshape(NC, N, 1)
    h1 = _dense_layer(p1, deg3, x, W1l, b1l, W1r, g1, be1, final=False)
    (p2,) = spmm(h1, meta, z2)
    h2 = _dense_layer(p2, deg3, h1, W2l, b2l, W2r, g2, be2, final=False)
    (p3,) = spmm(h2, meta, z2)
    out = _dense_layer(p3, deg3, h2, W3l, b3l, W3r, g1, be1, final=True)
    return out


# speedup vs baseline: 2.2003x; 2.2003x over previous
import functools

import jax
import jax.numpy as jnp
from jax import lax
from jax.experimental import pallas as pl
from jax.experimental.pallas import tpu as pltpu
from jax.experimental.pallas import tpu_sc as plsc

NC = 2
NS = 16
NW = NC * NS
CH = 128
LRELU = 0.1
BN_EPS = 1e-5
TRASH = 16


def _num_chunks(E):
    n = -(-E // (NW * CH))
    return n + (n % 2)


def _make_sc_spmm(N, D, E, with_deg):
    npc = _num_chunks(E)
    NP = NS * 640

    mesh = plsc.VectorSubcoreMesh(
        core_axis_name="c", subcore_axis_name="s",
        num_cores=NC, num_subcores=NS)

    out_type = [jax.ShapeDtypeStruct((NC, N, D), jnp.float32)]
    if with_deg:
        out_type.append(jax.ShapeDtypeStruct((NC, NP), jnp.float32))

    scratch = [
        pltpu.VMEM_SHARED((N + TRASH, D), jnp.float32),
        pltpu.VMEM_SHARED((NP,), jnp.float32),
        pltpu.VMEM((2, 128), jnp.int32),
        pltpu.VMEM((2, 128), jnp.int32),
        pltpu.VMEM((16, 128), jnp.float32),
        pltpu.VMEM((16, 128), jnp.float32),
        pltpu.VMEM((CH, D), jnp.float32),
        pltpu.VMEM((CH, D), jnp.float32),
        pltpu.VMEM((1, 128), jnp.int32),
        pltpu.VMEM((1, 128), jnp.int32),
        pltpu.VMEM((CH,), jnp.float32),
        pltpu.VMEM((640,), jnp.float32),
        pltpu.SemaphoreType.DMA,
        pltpu.SemaphoreType.DMA,
        pltpu.SemaphoreType.DMA,
        pltpu.SemaphoreType.DMA,
        pltpu.SemaphoreType.DMA,
        pltpu.SemaphoreType.DMA,
    ]

    @functools.partial(pl.kernel, out_type=tuple(out_type), mesh=mesh,
                       scratch_types=scratch)
    def spmm(*refs):
        if with_deg:
            (h_hbm, ids_hbm, w16_hbm, z2_hbm, out_hbm, deg_hbm, acc, deg_sh,
             ids0, ids1, wv0, wv1, rows0, rows1, dsc0, dsc1, ones_v, zer_v,
             sem_m0, sem_m1, sem_g0, sem_g1, sem_s0, sem_s1) = refs
        else:
            (h_hbm, ids_hbm, w16_hbm, z2_hbm, out_hbm, acc, deg_sh,
             ids0, ids1, wv0, wv1, rows0, rows1, dsc0, dsc1, ones_v, zer_v,
             sem_m0, sem_m1, sem_g0, sem_g1, sem_s0, sem_s1) = refs
            deg_hbm = None

        c = lax.axis_index("c")
        s = lax.axis_index("s")
        wid = s * NC + c
        ids = (ids0, ids1)
        wv = (wv0, wv1)
        rows = (rows0, rows1)
        dsc = (dsc0, dsc1)
        sem_m = (sem_m0, sem_m1)
        sem_g = (sem_g0, sem_g1)
        sem_s = (sem_s0, sem_s1)

        @pl.when(s < NS - 1)
        def _():
            pltpu.sync_copy(z2_hbm.at[pl.ds(s * 640, 640)],
                            acc.at[pl.ds(s * 640, 640)])

        @pl.when(s == NS - 1)
        def _():
            pltpu.sync_copy(z2_hbm.at[pl.ds(9600, 400)],
                            acc.at[pl.ds(9600, 400)])

        if with_deg:
            for i in range(640 // 16):
                zer_v[pl.ds(i * 16, 16)] = jnp.zeros((16,), jnp.float32)
            pltpu.sync_copy(zer_v, deg_sh.at[pl.ds(s * 640, 640)])
            for i in range(CH // 16):
                ones_v[pl.ds(i * 16, 16)] = jnp.ones((16,), jnp.float32)

        plsc.subcore_barrier()

        c0 = wid * npc
        clast = c0 + npc - 1

        def ids_start(ci, p):
            ci = jnp.minimum(ci, clast)
            pltpu.async_copy(ids_hbm.at[ci], ids[p], sem_m[p])

        def w16_start(ci, p):
            ci = jnp.minimum(ci, clast)
            pltpu.async_copy(w16_hbm.at[ci], wv[p], sem_m[p])

        def meta_wait(p):
            pltpu.make_async_copy(ids_hbm.at[c0], ids[p], sem_m[p]).wait()
            pltpu.make_async_copy(w16_hbm.at[c0], wv[p], sem_m[p]).wait()

        def cpy_dst(p):
            for k in range(128 // 16):
                sl = pl.ds(k * 16, 16)
                dsc[p][0, sl] = ids[p][1, sl]

        def gather_start(p):
            pltpu.async_copy(h_hbm.at[ids[p].at[0]], rows[p], sem_g[p])

        def gather_wait(p):
            pltpu.make_async_copy(h_hbm.at[ids[p].at[0]], rows[p],
                                  sem_g[p]).wait()

        def scale(p):
            rp, wp = rows[p], wv[p]

            def per_edge(e, carry2):
                bc = wp[e // 8, pl.ds((e % 8) * 16, 16)]
                for k in range(D // 16):
                    sl = pl.ds(k * 16, 16)
                    rp[e, sl] = rp[e, sl] * bc
                return carry2

            lax.fori_loop(0, CH, per_edge, 0)

        def scatter_start(p):
            pltpu.async_copy(rows[p], acc.at[dsc[p].at[0]], sem_s[p],
                             add=True)
            if with_deg:
                pltpu.async_copy(ones_v, deg_sh.at[dsc[p].at[0]], sem_s[p],
                                 add=True)

        def scatter_wait(p):
            pltpu.make_async_copy(rows[p], acc.at[dsc[p].at[0]],
                                  sem_s[p]).wait()
            if with_deg:
                pltpu.make_async_copy(ones_v, deg_sh.at[dsc[p].at[0]],
                                      sem_s[p]).wait()

        ids_start(c0, 0)
        w16_start(c0, 0)
        meta_wait(0)
        gather_start(0)
        ids_start(c0 + 1, 1)
        w16_start(c0 + 1, 1)

        def body(t, carry):
            a = c0 + 2 * t
            @pl.when(t > 0)
            def _():
                scatter_wait(1)
            meta_wait(1)
            gather_start(1)
            gather_wait(0)
            cpy_dst(0)
            ids_start(a + 2, 0)
            scale(0)
            scatter_start(0)
            w16_start(a + 2, 0)
            scatter_wait(0)
            meta_wait(0)
            gather_start(0)
            gather_wait(1)
            cpy_dst(1)
            ids_start(a + 3, 1)
            scale(1)
            scatter_start(1)
            w16_start(a + 3, 1)
            return carry

        lax.fori_loop(0, npc // 2, body, 0)
        scatter_wait(1)
        meta_wait(1)
        gather_wait(0)

        plsc.subcore_barrier()

        @pl.when(s < NS - 1)
        def _():
            pltpu.sync_copy(acc.at[pl.ds(s * 640, 640)],
                            out_hbm.at[c, pl.ds(s * 640, 640)])

        @pl.when(s == NS - 1)
        def _():
            pltpu.sync_copy(acc.at[pl.ds(9600, 400)],
                            out_hbm.at[c, pl.ds(9600, 400)])

        if with_deg:
            pltpu.sync_copy(deg_sh.at[pl.ds(s * 640, 640)],
                            deg_hbm.at[c, pl.ds(s * 640, 640)])

    return spmm


def _dense_layer(p, deg3, h, Wl, bl, Wr, gamma, beta, final):
    N, D = h.shape
    R = 2000
    grid = (N // R,)

    def body(p_ref, deg_ref, h_ref, wl_ref, bl_ref, wr_ref, g_ref, b_ref,
             o_ref):
        pb = p_ref[0] + p_ref[1]
        degb = deg_ref[0, :, 0] + deg_ref[1, :, 0]
        inv = 1.0 / jnp.maximum(degb, 1.0)
        agg = pb * inv[:, None]
        hh = (jnp.dot(agg, wl_ref[...], preferred_element_type=jnp.float32)
              + bl_ref[...]
              + jnp.dot(h_ref[...], wr_ref[...],
                        preferred_element_type=jnp.float32))
        if final:
            nrm = jnp.sqrt(jnp.sum(hh * hh, axis=1, keepdims=True))
            o_ref[...] = hh / jnp.maximum(nrm, 1e-12)
        else:
            scale = g_ref[...] * (1.0 / jnp.sqrt(1.0 + BN_EPS))
            hh = hh * scale + b_ref[...]
            o_ref[...] = jnp.where(hh >= 0, hh, LRELU * hh)

    return pl.pallas_call(
        body,
        grid=grid,
        in_specs=[
            pl.BlockSpec((NC, R, D), lambda i: (0, i, 0)),
            pl.BlockSpec((NC, R, 1), lambda i: (0, i, 0)),
            pl.BlockSpec((R, D), lambda i: (i, 0)),
            pl.BlockSpec((D, D), lambda i: (0, 0)),
            pl.BlockSpec((1, D), lambda i: (0, 0)),
            pl.BlockSpec((D, D), lambda i: (0, 0)),
            pl.BlockSpec((1, D), lambda i: (0, 0)),
            pl.BlockSpec((1, D), lambda i: (0, 0)),
        ],
        out_specs=pl.BlockSpec((R, D), lambda i: (i, 0)),
        out_shape=jax.ShapeDtypeStruct((N, D), jnp.float32),
    )(p, deg3, h, Wl, bl.reshape(1, D), Wr, gamma.reshape(1, D),
      beta.reshape(1, D))


def kernel(x, edge_index, edge_weight, W1l, b1l, W1r, W2l, b2l, W2r,
           W3l, b3l, W3r, g1, be1, g2, be2):
    N, D = x.shape
    E = edge_weight.shape[0]
    Ep = _num_chunks(E) * NW * CH

    pad = Ep - E
    src = jnp.concatenate([edge_index[0], jnp.zeros((pad,), jnp.int32)])
    tr = N + (jnp.arange(pad, dtype=jnp.int32) % TRASH)
    dst = jnp.concatenate([edge_index[1], tr])
    w = jnp.concatenate([edge_weight, jnp.zeros((pad,), jnp.float32)])

    nck = Ep // CH
    w16 = jnp.repeat(w, 16).reshape(nck, 16, 128)
    ids = jnp.concatenate(
        [src.reshape(nck, 1, 128), dst.reshape(nck, 1, 128)], axis=1)

    z2 = jnp.zeros((N, D), jnp.float32)

    spmm_deg = _make_sc_spmm(N, D, Ep, with_deg=True)
    spmm = _make_sc_spmm(N, D, Ep, with_deg=False)

    p1, deg = spmm_deg(x, ids, w16, z2)
    deg3 = deg[:, :N].reshape(NC, N, 1)
    h1 = _dense_layer(p1, deg3, x, W1l, b1l, W1r, g1, be1, final=False)
    (p2,) = spmm(h1, ids, w16, z2)
    h2 = _dense_layer(p2, deg3, h1, W2l, b2l, W2r, g2, be2, final=False)
    (p3,) = spmm(h2, ids, w16, z2)
    out = _dense_layer(p3, deg3, h2, W3l, b3l, W3r, g1, be1, final=True)
    return out


# speedup vs baseline: 3.2205x; 1.4636x over previous
import functools

import jax
import jax.numpy as jnp
from jax import lax
from jax.experimental import pallas as pl
from jax.experimental.pallas import tpu as pltpu
from jax.experimental.pallas import tpu_sc as plsc

NC = 2
NS = 16
NW = NC * NS
LRELU = 0.1
BN_EPS = 1e-5


def _make_sc_spmm(N, D, E, with_deg):
    CH = 256
    nfull = E // (NW * CH)
    nextra = E // CH - NW * nfull
    assert E == (NW * nfull + nextra) * CH
    NP = NS * 640

    mesh = plsc.VectorSubcoreMesh(
        core_axis_name="c", subcore_axis_name="s",
        num_cores=NC, num_subcores=NS)

    out_type = [jax.ShapeDtypeStruct((NC, N, D), jnp.float32)]
    if with_deg:
        out_type.append(jax.ShapeDtypeStruct((NC, NP), jnp.float32))

    scratch = [
        pltpu.VMEM_SHARED((N, D), jnp.float32),
        pltpu.VMEM_SHARED((NP,), jnp.float32),
        pltpu.VMEM((CH,), jnp.int32),
        pltpu.VMEM((CH,), jnp.int32),
        pltpu.VMEM((CH // 8, 128), jnp.float32),
        pltpu.VMEM((CH, D), jnp.float32),
        pltpu.VMEM((CH,), jnp.float32),
        pltpu.VMEM((640,), jnp.float32),
        pltpu.SemaphoreType.DMA,
    ]

    @functools.partial(pl.kernel, out_type=tuple(out_type), mesh=mesh,
                       scratch_types=scratch)
    def spmm(*refs):
        if with_deg:
            (h_hbm, src_hbm, dst_hbm, w_hbm, z2_hbm,
             out_hbm, deg_hbm, acc, deg_sh,
             src_v, dst_v, w_v, rows_v, ones_v, zer_v, sem) = refs
        else:
            (h_hbm, src_hbm, dst_hbm, w_hbm, z2_hbm,
             out_hbm, acc, deg_sh,
             src_v, dst_v, w_v, rows_v, ones_v, zer_v, sem) = refs
            deg_hbm = None

        c = lax.axis_index("c")
        s = lax.axis_index("s")
        wid = s * NC + c

        @pl.when(s < NS - 1)
        def _():
            pltpu.sync_copy(z2_hbm.at[pl.ds(s * 640, 640)],
                            acc.at[pl.ds(s * 640, 640)])

        @pl.when(s == NS - 1)
        def _():
            pltpu.sync_copy(z2_hbm.at[pl.ds(9600, 400)],
                            acc.at[pl.ds(9600, 400)])

        if with_deg:
            for i in range(640 // 16):
                zer_v[pl.ds(i * 16, 16)] = jnp.zeros((16,), jnp.float32)
            pltpu.sync_copy(zer_v, deg_sh.at[pl.ds(s * 640, 640)])
            for i in range(CH // 16):
                ones_v[pl.ds(i * 16, 16)] = jnp.ones((16,), jnp.float32)

        plsc.subcore_barrier()

        base0 = nfull * CH * wid + CH * jnp.minimum(wid, nextra)
        nch = nfull + (wid < nextra).astype(jnp.int32)

        def chunk(i, carry):
            base = pl.multiple_of(base0 + i * CH, 256)
            pltpu.sync_copy(src_hbm.at[pl.ds(base, CH)], src_v)
            pltpu.sync_copy(dst_hbm.at[pl.ds(base, CH)], dst_v)
            pltpu.sync_copy(
                w_hbm.at[pl.ds(pl.multiple_of(base // 8, 8), CH // 8)], w_v)
            pltpu.async_copy(h_hbm.at[src_v], rows_v, sem).wait()

            def grp(g, carry2):
                for j in range(16):
                    e = g * 16 + j
                    bc = w_v[2 * g + (j // 8), pl.ds((j % 8) * 16, 16)]
                    for k in range(D // 16):
                        sl = pl.ds(k * 16, 16)
                        rows_v[e, sl] = rows_v[e, sl] * bc
                return carry2

            lax.fori_loop(0, CH // 16, grp, 0)

            pltpu.sync_copy(rows_v, acc.at[dst_v], add=True)
            if with_deg:
                pltpu.sync_copy(ones_v, deg_sh.at[dst_v], add=True)
            return carry

        lax.fori_loop(0, nch, chunk, 0)

        plsc.subcore_barrier()

        @pl.when(s < NS - 1)
        def _():
            pltpu.sync_copy(acc.at[pl.ds(s * 640, 640)],
                            out_hbm.at[c, pl.ds(s * 640, 640)])

        @pl.when(s == NS - 1)
        def _():
            pltpu.sync_copy(acc.at[pl.ds(9600, 400)],
                            out_hbm.at[c, pl.ds(9600, 400)])

        if with_deg:
            pltpu.sync_copy(deg_sh.at[pl.ds(s * 640, 640)],
                            deg_hbm.at[c, pl.ds(s * 640, 640)])

    return spmm


def _dense_layer(p, deg3, h, Wl, bl, Wr, gamma, beta, final):
    N, D = h.shape
    R = 2000
    grid = (N // R,)

    def body(p_ref, deg_ref, h_ref, wl_ref, bl_ref, wr_ref, g_ref, b_ref,
             o_ref):
        pb = p_ref[0] + p_ref[1]
        degb = deg_ref[0, :, 0] + deg_ref[1, :, 0]
        inv = 1.0 / jnp.maximum(degb, 1.0)
        agg = pb * inv[:, None]
        hh = (jnp.dot(agg, wl_ref[...], preferred_element_type=jnp.float32)
              + bl_ref[...]
              + jnp.dot(h_ref[...], wr_ref[...],
                        preferred_element_type=jnp.float32))
        if final:
            nrm = jnp.sqrt(jnp.sum(hh * hh, axis=1, keepdims=True))
            o_ref[...] = hh / jnp.maximum(nrm, 1e-12)
        else:
            scale = g_ref[...] * (1.0 / jnp.sqrt(1.0 + BN_EPS))
            hh = hh * scale + b_ref[...]
            o_ref[...] = jnp.where(hh >= 0, hh, LRELU * hh)

    return pl.pallas_call(
        body,
        grid=grid,
        in_specs=[
            pl.BlockSpec((NC, R, D), lambda i: (0, i, 0)),
            pl.BlockSpec((NC, R, 1), lambda i: (0, i, 0)),
            pl.BlockSpec((R, D), lambda i: (i, 0)),
            pl.BlockSpec((D, D), lambda i: (0, 0)),
            pl.BlockSpec((1, D), lambda i: (0, 0)),
            pl.BlockSpec((D, D), lambda i: (0, 0)),
            pl.BlockSpec((1, D), lambda i: (0, 0)),
            pl.BlockSpec((1, D), lambda i: (0, 0)),
        ],
        out_specs=pl.BlockSpec((R, D), lambda i: (i, 0)),
        out_shape=jax.ShapeDtypeStruct((N, D), jnp.float32),
    )(p, deg3, h, Wl, bl.reshape(1, D), Wr, gamma.reshape(1, D),
      beta.reshape(1, D))


def kernel(x, edge_index, edge_weight, W1l, b1l, W1r, W2l, b2l, W2r,
           W3l, b3l, W3r, g1, be1, g2, be2):
    N, D = x.shape
    E = edge_weight.shape[0]
    src = edge_index[0]
    dst = edge_index[1]
    w128 = jnp.repeat(edge_weight, 16).reshape(E // 8, 128)
    z2 = jnp.zeros((N, D), jnp.float32)

    spmm_deg = _make_sc_spmm(N, D, E, with_deg=True)
    spmm = _make_sc_spmm(N, D, E, with_deg=False)

    p1, deg = spmm_deg(x, src, dst, w128, z2)
    deg3 = deg[:, :N].reshape(NC, N, 1)
    h1 = _dense_layer(p1, deg3, x, W1l, b1l, W1r, g1, be1, final=False)
    (p2,) = spmm(h1, src, dst, w128, z2)
    h2 = _dense_layer(p2, deg3, h1, W2l, b2l, W2r, g2, be2, final=False)
    (p3,) = spmm(h2, src, dst, w128, z2)
    out = _dense_layer(p3, deg3, h2, W3l, b3l, W3r, g1, be1, final=True)
    return out
